# baseline (device time: 164294 ns/iter reference)
import jax
import jax.numpy as jnp
from jax import lax
from jax.experimental import pallas as pl
from jax.experimental.pallas import tpu as pltpu

N = 4
B = 4
S = 1024
HD = 512
E = 2048
SC = S // N


def kernel(O, Wo):
    O3 = O.reshape(B, S, HD)

    def body(o_ref, w_ref, out_ref, comm_ref, wb_ref, send_sems, recv_sems):
        my_x = lax.axis_index("x")
        my_y = lax.axis_index("y")
        my_z = lax.axis_index("z")
        left_y = (my_y + N - 1) % N
        right_y = (my_y + 1) % N

        barrier_sem = pltpu.get_barrier_semaphore()
        for nbr in (left_y, right_y):
            pl.semaphore_signal(
                barrier_sem, inc=1,
                device_id=(my_x, nbr, my_z),
                device_id_type=pl.DeviceIdType.MESH,
            )
        pl.semaphore_wait(barrier_sem, 2)

        wb_ref[...] = w_ref[...].astype(jnp.bfloat16)

        def partial(b, j):
            a = o_ref[b, pl.ds(j * SC, SC), :].astype(jnp.bfloat16)
            return jnp.dot(a, wb_ref[...], preferred_element_type=jnp.float32)

        j0 = (my_y + N - 1) % N
        for b in range(B):
            comm_ref[0, b] = partial(b, j0).astype(jnp.bfloat16)

        for t in range(N - 1):
            rdma = pltpu.make_async_remote_copy(
                src_ref=comm_ref.at[t],
                dst_ref=comm_ref.at[t + 1],
                send_sem=send_sems.at[t],
                recv_sem=recv_sems.at[t],
                device_id=(my_x, right_y, my_z),
                device_id_type=pl.DeviceIdType.MESH,
            )
            rdma.start()
            rdma.wait()

            r = (my_y + 2 * N - 2 - t) % N
            for b in range(B):
                tot = partial(b, r) + comm_ref[t + 1, b].astype(jnp.float32)
                if t < N - 2:
                    comm_ref[t + 1, b] = tot.astype(jnp.bfloat16)
                else:
                    out_ref[b] = tot

    return pl.pallas_call(
        body,
        out_shape=jax.ShapeDtypeStruct((B, SC, E), jnp.float32),
        in_specs=[
            pl.BlockSpec(memory_space=pltpu.VMEM),
            pl.BlockSpec(memory_space=pltpu.VMEM),
        ],
        out_specs=pl.BlockSpec(memory_space=pltpu.VMEM),
        scratch_shapes=[
            pltpu.VMEM((N, B, SC, E), jnp.bfloat16),
            pltpu.VMEM((HD, E), jnp.bfloat16),
            pltpu.SemaphoreType.DMA((N - 1,)),
            pltpu.SemaphoreType.DMA((N - 1,)),
        ],
        compiler_params=pltpu.CompilerParams(collective_id=0),
    )(O3, Wo)


# device time: 74125 ns/iter; 2.2164x vs baseline; 2.2164x over previous
import jax
import jax.numpy as jnp
from jax import lax
from jax.experimental import pallas as pl
from jax.experimental.pallas import tpu as pltpu

NY = 4
NXZ = 8
B = 4
S = 1024
HD = 512
E = 2048
SC = S // NY
EC = E // NXZ
CW, CCW = 4, 3


def kernel(O, Wo):
    O3 = O.reshape(B, S, HD)
    WoT = Wo.reshape(HD, NXZ, EC).transpose(1, 0, 2)

    def body(o_ref, w_ref, out_ref, comm_ref, wb_ref, ag_ref,
             y_send, y_recv, cw_send, cw_recv, ccw_send, ccw_recv):
        x = lax.axis_index("x")
        y = lax.axis_index("y")
        z = lax.axis_index("z")

        m = jnp.where(x == 0, z, 7 - z)

        def ring_coords(pos):
            rx = jnp.where(pos < 4, 0, 1)
            rz = jnp.where(pos < 4, pos, 7 - pos)
            return rx, rz

        yl = (y + NY - 1) % NY
        yr = (y + 1) % NY
        rr_x, rr_z = ring_coords((m + 1) % NXZ)
        rl_x, rl_z = ring_coords((m + NXZ - 1) % NXZ)
        cw_dev = (rr_x, y, rr_z)
        ccw_dev = (rl_x, y, rl_z)

        barrier = pltpu.get_barrier_semaphore()
        for dev in ((x, yl, z), (x, yr, z), cw_dev, ccw_dev):
            pl.semaphore_signal(barrier, inc=1, device_id=dev,
                                device_id_type=pl.DeviceIdType.MESH)
        pl.semaphore_wait(barrier, 4)

        wb_ref[...] = w_ref[m].astype(jnp.bfloat16)

        def partial(b, j):
            a = o_ref[b, pl.ds(j * SC, SC), :].astype(jnp.bfloat16)
            return jnp.dot(a, wb_ref[...], preferred_element_type=jnp.float32)

        j0 = (y + NY - 1) % NY
        for b in range(B):
            comm_ref[0, b] = partial(b, j0).astype(jnp.bfloat16)
        for t in range(NY - 1):
            rdma = pltpu.make_async_remote_copy(
                src_ref=comm_ref.at[t], dst_ref=comm_ref.at[t + 1],
                send_sem=y_send.at[t], recv_sem=y_recv.at[t],
                device_id=(x, yr, z), device_id_type=pl.DeviceIdType.MESH)
            rdma.start()
            rdma.wait()
            r = (y + 2 * NY - 2 - t) % NY
            for b in range(B):
                tot = partial(b, r) + comm_ref[t + 1, b].astype(jnp.float32)
                if t < NY - 2:
                    comm_ref[t + 1, b] = tot.astype(jnp.bfloat16)
                else:
                    ag_ref[m, b] = tot.astype(jnp.bfloat16)

        def mk(slot, send_sem, recv_sem, dev):
            return pltpu.make_async_remote_copy(
                src_ref=ag_ref.at[slot], dst_ref=ag_ref.at[slot],
                send_sem=send_sem, recv_sem=recv_sem,
                device_id=dev, device_id_type=pl.DeviceIdType.MESH)

        sends = []
        d = mk(m, cw_send.at[0], cw_recv.at[0], cw_dev)
        d.start()
        sends.append(d)
        d = mk(m, ccw_send.at[0], ccw_recv.at[0], ccw_dev)
        d.start()
        sends.append(d)
        for h in range(CW):
            slot = (m + 2 * NXZ - 1 - h) % NXZ
            mk(slot, cw_send.at[h], cw_recv.at[h], cw_dev).wait_recv()
            if h + 1 < CW:
                d = mk(slot, cw_send.at[h + 1], cw_recv.at[h + 1], cw_dev)
                d.start()
                sends.append(d)
            if h < CCW:
                slot2 = (m + 1 + h) % NXZ
                mk(slot2, ccw_send.at[h], ccw_recv.at[h], ccw_dev).wait_recv()
                if h + 1 < CCW:
                    d = mk(slot2, ccw_send.at[h + 1], ccw_recv.at[h + 1],
                           ccw_dev)
                    d.start()
                    sends.append(d)
        for d in sends:
            d.wait_send()

        for k in range(NXZ):
            for b in range(B):
                out_ref[b, :, k * EC:(k + 1) * EC] = (
                    ag_ref[k, b].astype(jnp.float32))

    return pl.pallas_call(
        body,
        out_shape=jax.ShapeDtypeStruct((B, SC, E), jnp.float32),
        in_specs=[
            pl.BlockSpec(memory_space=pltpu.VMEM),
            pl.BlockSpec(memory_space=pltpu.VMEM),
        ],
        out_specs=pl.BlockSpec(memory_space=pltpu.VMEM),
        scratch_shapes=[
            pltpu.VMEM((NY, B, SC, EC), jnp.bfloat16),
            pltpu.VMEM((HD, EC), jnp.bfloat16),
            pltpu.VMEM((NXZ, B, SC, EC), jnp.bfloat16),
            pltpu.SemaphoreType.DMA((NY - 1,)),
            pltpu.SemaphoreType.DMA((NY - 1,)),
            pltpu.SemaphoreType.DMA((CW,)),
            pltpu.SemaphoreType.DMA((CW,)),
            pltpu.SemaphoreType.DMA((CCW,)),
            pltpu.SemaphoreType.DMA((CCW,)),
        ],
        compiler_params=pltpu.CompilerParams(collective_id=0),
    )(O3, WoT)


# device time: 73666 ns/iter; 2.2303x vs baseline; 1.0062x over previous
import jax
import jax.numpy as jnp
from jax import lax
from jax.experimental import pallas as pl
from jax.experimental.pallas import tpu as pltpu

NY = 4
NXZ = 8
B = 4
S = 1024
HD = 512
E = 2048
SC = S // NY
EC = E // NXZ
CW, CCW = 4, 3


def kernel(O, Wo):
    O3 = O.reshape(B, S, HD)
    WoT = Wo.reshape(HD, NXZ, EC).transpose(1, 0, 2)

    def body(o_ref, w_ref, out_ref, comm_ref, wb_ref, ag_ref,
             y_send, y_recv, cw_send, cw_recv, ccw_send, ccw_recv):
        x = lax.axis_index("x")
        y = lax.axis_index("y")
        z = lax.axis_index("z")

        m = jnp.where(x == 0, z, 7 - z)

        def ring_coords(pos):
            rx = jnp.where(pos < 4, 0, 1)
            rz = jnp.where(pos < 4, pos, 7 - pos)
            return rx, rz

        yl = (y + NY - 1) % NY
        yr = (y + 1) % NY
        rr_x, rr_z = ring_coords((m + 1) % NXZ)
        rl_x, rl_z = ring_coords((m + NXZ - 1) % NXZ)
        cw_dev = (rr_x, y, rr_z)
        ccw_dev = (rl_x, y, rl_z)

        barrier = pltpu.get_barrier_semaphore()
        for dev in ((x, yl, z), (x, yr, z), cw_dev, ccw_dev):
            pl.semaphore_signal(barrier, inc=1, device_id=dev,
                                device_id_type=pl.DeviceIdType.MESH)

        wb_ref[...] = w_ref[m].astype(jnp.bfloat16)

        def partial(b, j):
            a = o_ref[b, pl.ds(j * SC, SC), :].astype(jnp.bfloat16)
            return jnp.dot(a, wb_ref[...], preferred_element_type=jnp.float32)

        j0 = (y + NY - 1) % NY
        for b in range(B):
            comm_ref[0, b] = partial(b, j0).astype(jnp.bfloat16)

        pl.semaphore_wait(barrier, 4)

        p1_sends = []
        for t in range(NY - 1):
            rdma = pltpu.make_async_remote_copy(
                src_ref=comm_ref.at[t], dst_ref=comm_ref.at[t + 1],
                send_sem=y_send.at[t], recv_sem=y_recv.at[t],
                device_id=(x, yr, z), device_id_type=pl.DeviceIdType.MESH)
            rdma.start()
            p1_sends.append(rdma)
            r = (y + 2 * NY - 2 - t) % NY
            parts = [partial(b, r) for b in range(B)]
            rdma.wait_recv()
            for b in range(B):
                tot = parts[b] + comm_ref[t + 1, b].astype(jnp.float32)
                if t < NY - 2:
                    comm_ref[t + 1, b] = tot.astype(jnp.bfloat16)
                else:
                    ag_ref[m, b] = tot.astype(jnp.bfloat16)

        def mk(slot, send_sem, recv_sem, dev):
            return pltpu.make_async_remote_copy(
                src_ref=ag_ref.at[slot], dst_ref=ag_ref.at[slot],
                send_sem=send_sem, recv_sem=recv_sem,
                device_id=dev, device_id_type=pl.DeviceIdType.MESH)

        sends = []
        d = mk(m, cw_send.at[0], cw_recv.at[0], cw_dev)
        d.start()
        sends.append(d)
        d = mk(m, ccw_send.at[0], ccw_recv.at[0], ccw_dev)
        d.start()
        sends.append(d)
        for h in range(CW):
            slot = (m + 2 * NXZ - 1 - h) % NXZ
            mk(slot, cw_send.at[h], cw_recv.at[h], cw_dev).wait_recv()
            if h + 1 < CW:
                d = mk(slot, cw_send.at[h + 1], cw_recv.at[h + 1], cw_dev)
                d.start()
                sends.append(d)
            if h < CCW:
                slot2 = (m + 1 + h) % NXZ
                mk(slot2, ccw_send.at[h], ccw_recv.at[h], ccw_dev).wait_recv()
                if h + 1 < CCW:
                    d = mk(slot2, ccw_send.at[h + 1], ccw_recv.at[h + 1],
                           ccw_dev)
                    d.start()
                    sends.append(d)
        for k in range(NXZ):
            for b in range(B):
                out_ref[b, :, k * EC:(k + 1) * EC] = (
                    ag_ref[k, b].astype(jnp.float32))

        for d in sends:
            d.wait_send()
        for d in p1_sends:
            d.wait_send()

    return pl.pallas_call(
        body,
        out_shape=jax.ShapeDtypeStruct((B, SC, E), jnp.float32),
        in_specs=[
            pl.BlockSpec(memory_space=pltpu.VMEM),
            pl.BlockSpec(memory_space=pltpu.VMEM),
        ],
        out_specs=pl.BlockSpec(memory_space=pltpu.VMEM),
        scratch_shapes=[
            pltpu.VMEM((NY, B, SC, EC), jnp.bfloat16),
            pltpu.VMEM((HD, EC), jnp.bfloat16),
            pltpu.VMEM((NXZ, B, SC, EC), jnp.bfloat16),
            pltpu.SemaphoreType.DMA((NY - 1,)),
            pltpu.SemaphoreType.DMA((NY - 1,)),
            pltpu.SemaphoreType.DMA((CW,)),
            pltpu.SemaphoreType.DMA((CW,)),
            pltpu.SemaphoreType.DMA((CCW,)),
            pltpu.SemaphoreType.DMA((CCW,)),
        ],
        compiler_params=pltpu.CompilerParams(collective_id=0),
    )(O3, WoT)


# device time: 73108 ns/iter; 2.2473x vs baseline; 1.0076x over previous
import jax
import jax.numpy as jnp
from jax import lax
from jax.experimental import pallas as pl
from jax.experimental.pallas import tpu as pltpu

NY = 4
NXZ = 8
B = 4
S = 1024
HD = 512
E = 2048
SC = S // NY
EC = E // NXZ
CW, CCW = 4, 3


def kernel(O, Wo):
    O3 = O.reshape(B, S, HD)
    WoT = Wo.reshape(HD, NXZ, EC).transpose(1, 0, 2)

    def body(o_ref, w_ref, out_ref, comm_ref, wb_ref, ag_ref,
             y_send, y_recv, cw_send, cw_recv, ccw_send, ccw_recv):
        x = lax.axis_index("x")
        y = lax.axis_index("y")
        z = lax.axis_index("z")

        m = jnp.where(x == 0, z, 7 - z)

        def ring_coords(pos):
            rx = jnp.where(pos < 4, 0, 1)
            rz = jnp.where(pos < 4, pos, 7 - pos)
            return rx, rz

        yl = (y + NY - 1) % NY
        yr = (y + 1) % NY
        rr_x, rr_z = ring_coords((m + 1) % NXZ)
        rl_x, rl_z = ring_coords((m + NXZ - 1) % NXZ)
        cw_dev = (rr_x, y, rr_z)
        ccw_dev = (rl_x, y, rl_z)

        barrier = pltpu.get_barrier_semaphore()
        for dev in ((x, yl, z), (x, yr, z), cw_dev, ccw_dev):
            pl.semaphore_signal(barrier, inc=1, device_id=dev,
                                device_id_type=pl.DeviceIdType.MESH)

        wb_ref[...] = w_ref[m].astype(jnp.bfloat16)

        def partial(b, j):
            a = o_ref[b, pl.ds(j * SC, SC), :].astype(jnp.bfloat16)
            return jnp.dot(a, wb_ref[...], preferred_element_type=jnp.float32)

        j0 = (y + NY - 1) % NY
        for b in range(B):
            comm_ref[0, b] = partial(b, j0).astype(jnp.bfloat16)

        pl.semaphore_wait(barrier, 4)

        p1_sends = []
        for t in range(NY - 1):
            rdma = pltpu.make_async_remote_copy(
                src_ref=comm_ref.at[t], dst_ref=comm_ref.at[t + 1],
                send_sem=y_send.at[t], recv_sem=y_recv.at[t],
                device_id=(x, yr, z), device_id_type=pl.DeviceIdType.MESH)
            rdma.start()
            p1_sends.append(rdma)
            r = (y + 2 * NY - 2 - t) % NY
            parts = [partial(b, r) for b in range(B)]
            rdma.wait_recv()
            for b in range(B):
                tot = parts[b] + comm_ref[t + 1, b].astype(jnp.float32)
                if t < NY - 2:
                    comm_ref[t + 1, b] = tot.astype(jnp.bfloat16)
                else:
                    ag_ref[m, b] = tot.astype(jnp.bfloat16)

        def mk(slot, send_sem, recv_sem, dev):
            return pltpu.make_async_remote_copy(
                src_ref=ag_ref.at[slot], dst_ref=ag_ref.at[slot],
                send_sem=send_sem, recv_sem=recv_sem,
                device_id=dev, device_id_type=pl.DeviceIdType.MESH)

        def repack(slot):
            for b in range(B):
                out_ref[b, :, pl.ds(slot * EC, EC)] = (
                    ag_ref[slot, b].astype(jnp.float32))

        sends = []
        d = mk(m, cw_send.at[0], cw_recv.at[0], cw_dev)
        d.start()
        sends.append(d)
        d = mk(m, ccw_send.at[0], ccw_recv.at[0], ccw_dev)
        d.start()
        sends.append(d)
        repack(m)
        for h in range(CW):
            slot = (m + 2 * NXZ - 1 - h) % NXZ
            mk(slot, cw_send.at[h], cw_recv.at[h], cw_dev).wait_recv()
            if h + 1 < CW:
                d = mk(slot, cw_send.at[h + 1], cw_recv.at[h + 1], cw_dev)
                d.start()
                sends.append(d)
            repack(slot)
            if h < CCW:
                slot2 = (m + 1 + h) % NXZ
                mk(slot2, ccw_send.at[h], ccw_recv.at[h], ccw_dev).wait_recv()
                if h + 1 < CCW:
                    d = mk(slot2, ccw_send.at[h + 1], ccw_recv.at[h + 1],
                           ccw_dev)
                    d.start()
                    sends.append(d)
                repack(slot2)

        for d in sends:
            d.wait_send()
        for d in p1_sends:
            d.wait_send()

    return pl.pallas_call(
        body,
        out_shape=jax.ShapeDtypeStruct((B, SC, E), jnp.float32),
        in_specs=[
            pl.BlockSpec(memory_space=pltpu.VMEM),
            pl.BlockSpec(memory_space=pltpu.VMEM),
        ],
        out_specs=pl.BlockSpec(memory_space=pltpu.VMEM),
        scratch_shapes=[
            pltpu.VMEM((NY, B, SC, EC), jnp.bfloat16),
            pltpu.VMEM((HD, EC), jnp.bfloat16),
            pltpu.VMEM((NXZ, B, SC, EC), jnp.bfloat16),
            pltpu.SemaphoreType.DMA((NY - 1,)),
            pltpu.SemaphoreType.DMA((NY - 1,)),
            pltpu.SemaphoreType.DMA((CW,)),
            pltpu.SemaphoreType.DMA((CW,)),
            pltpu.SemaphoreType.DMA((CCW,)),
            pltpu.SemaphoreType.DMA((CCW,)),
        ],
        compiler_params=pltpu.CompilerParams(collective_id=0),
    )(O3, WoT)
